# Initial kernel scaffold; baseline (speedup 1.0000x reference)
import jax
import jax.numpy as jnp
from jax.experimental import pallas as pl


def kernel(charges, d_ij_lr, nbmat_lr):
    raise NotImplementedError("write your pallas kernel here")


# speedup vs baseline: 7.3660x; 7.3660x over previous
import functools

import jax
import jax.numpy as jnp
from jax import lax
from jax.experimental import pallas as pl
from jax.experimental.pallas import tpu as pltpu
from jax.experimental.pallas import tpu_sc as plsc

jax.config.update("jax_enable_x64", True)

_FACTOR = 13.605693122994 * 0.5291772105638411

_NC = 2
_NS = 16
_NW = _NC * _NS
_L = 16


def _make_sc_call(N, M):
    RPW = N // _NW
    CR = 25
    assert RPW % CR == 0
    NCH = RPW // CR
    PM = 2 * M
    NV = M // _L

    mesh = plsc.VectorSubcoreMesh(core_axis_name="c", subcore_axis_name="s")

    @functools.partial(
        pl.kernel,
        out_type=jax.ShapeDtypeStruct((_NW, _L), jnp.float32),
        mesh=mesh,
        scratch_types=[
            pltpu.VMEM((N + _L,), jnp.float32),
            pltpu.VMEM((CR * PM,), jnp.int32),
            pltpu.VMEM((CR * PM,), jnp.int32),
            pltpu.VMEM((CR * M,), jnp.float32),
            pltpu.VMEM((CR * M,), jnp.float32),
            pltpu.VMEM((_L,), jnp.float32),
            pltpu.SemaphoreType.DMA,
            pltpu.SemaphoreType.DMA,
            pltpu.SemaphoreType.DMA,
        ],
        compiler_params=pltpu.CompilerParams(needs_layout_passes=False),
    )
    def sc_fn(q_hbm, d_hbm, nb_hbm, out_hbm,
              q_v, nb_v0, nb_v1, d_v0, d_v1, acc_v, sem0, sem1, semt):
        wid = lax.axis_index("s") * _NC + lax.axis_index("c")
        base_row = wid * RPW
        nb_bufs = (nb_v0, nb_v1)
        d_bufs = (d_v0, d_v1)
        sems = (sem0, sem1)

        pltpu.async_copy(q_hbm, q_v.at[pl.ds(0, N)], semt)

        iota = lax.iota(jnp.int32, _L)

        def start_chunk(c, b):
            row0 = base_row + c * CR
            pltpu.async_copy(nb_hbm.at[pl.ds(row0 * PM, CR * PM)],
                             nb_bufs[b], sems[b])
            pltpu.async_copy(d_hbm.at[pl.ds(row0 * M, CR * M)],
                             d_bufs[b], sems[b])

        def wait_chunk(b):
            pltpu.make_async_copy(nb_hbm.at[pl.ds(0, CR * PM)],
                                  nb_bufs[b], sems[b]).wait()
            pltpu.make_async_copy(d_hbm.at[pl.ds(0, CR * M)],
                                  d_bufs[b], sems[b]).wait()

        def process_chunk(c, b, acc):
            nb_v = nb_bufs[b]
            d_v = d_bufs[b]

            def row_body(r, acc):
                row_sum = jnp.zeros((_L,), jnp.float32)
                for v in range(NV):
                    idx = plsc.load_gather(
                        nb_v, [r * PM + (2 * v * _L) + 2 * iota])
                    qj = plsc.load_gather(q_v, [idx])
                    dv = plsc.load_gather(d_v, [r * M + v * _L + iota])
                    row_sum = row_sum + qj / dv
                qi = q_v[pl.ds(base_row + c * CR + r, _L)][0]
                return acc + qi * row_sum

            return lax.fori_loop(jnp.int32(0), jnp.int32(CR), row_body, acc,
                                 unroll=False)

        start_chunk(0, 0)
        pltpu.make_async_copy(q_hbm, q_v.at[pl.ds(0, N)], semt).wait()

        def body(k, acc):
            g = 2 * k
            start_chunk(g + 1, 1)
            wait_chunk(0)
            acc = process_chunk(g, 0, acc)
            start_chunk(g + 2, 0)
            wait_chunk(1)
            acc = process_chunk(g + 1, 1, acc)
            return acc

        acc = lax.fori_loop(jnp.int32(0), jnp.int32((NCH - 1) // 2),
                            body, jnp.zeros((_L,), jnp.float32))
        wait_chunk(0)
        acc = process_chunk(NCH - 1, 0, acc)

        acc_v[...] = acc
        pltpu.sync_copy(acc_v, out_hbm.at[wid])

    return sc_fn


@jax.jit
def kernel(charges, d_ij_lr, nbmat_lr):
    B, N, M = d_ij_lr.shape
    q = charges.reshape(N)
    d = d_ij_lr.reshape(N * M)
    nb = lax.bitcast_convert_type(nbmat_lr, jnp.int32).reshape(N * M * 2)
    partials = _make_sc_call(N, M)(q, d, nb)
    e = _FACTOR * jnp.sum(partials.astype(jnp.float64))
    return e.reshape(B)


# speedup vs baseline: 7.3676x; 1.0002x over previous
import functools

import jax
import jax.numpy as jnp
from jax import lax
from jax.experimental import pallas as pl
from jax.experimental.pallas import tpu as pltpu
from jax.experimental.pallas import tpu_sc as plsc

jax.config.update("jax_enable_x64", True)

_FACTOR = 13.605693122994 * 0.5291772105638411

_NC = 2
_NS = 16
_NW = _NC * _NS
_L = 16


def _make_sc_call(N, M):
    RPW = N // _NW
    CR = 25
    assert RPW % CR == 0
    NCH = RPW // CR
    PM = 2 * M
    NV = M // _L

    mesh = plsc.VectorSubcoreMesh(core_axis_name="c", subcore_axis_name="s")

    @functools.partial(
        pl.kernel,
        out_type=jax.ShapeDtypeStruct((_NW, _L), jnp.float32),
        mesh=mesh,
        scratch_types=[
            pltpu.VMEM((N + _L,), jnp.float32),
            pltpu.VMEM((CR * PM,), jnp.int32),
            pltpu.VMEM((CR * PM,), jnp.int32),
            pltpu.VMEM((CR * M,), jnp.float32),
            pltpu.VMEM((CR * M,), jnp.float32),
            pltpu.VMEM((_L,), jnp.float32),
            pltpu.SemaphoreType.DMA,
            pltpu.SemaphoreType.DMA,
            pltpu.SemaphoreType.DMA,
        ],
        compiler_params=pltpu.CompilerParams(needs_layout_passes=False,
                                             use_tc_tiling_on_sc=True),
    )
    def sc_fn(q_hbm, d_hbm, nb_hbm, out_hbm,
              q_v, nb_v0, nb_v1, d_v0, d_v1, acc_v, sem0, sem1, semt):
        wid = lax.axis_index("s") * _NC + lax.axis_index("c")
        base_row = wid * RPW
        nb_bufs = (nb_v0, nb_v1)
        d_bufs = (d_v0, d_v1)
        sems = (sem0, sem1)

        pltpu.async_copy(q_hbm, q_v.at[pl.ds(0, N)], semt)

        iota = lax.iota(jnp.int32, _L)

        def start_chunk(c, b):
            row0 = base_row + c * CR
            pltpu.async_copy(nb_hbm.at[pl.ds(row0 * PM, CR * PM)],
                             nb_bufs[b], sems[b])
            pltpu.async_copy(d_hbm.at[pl.ds(row0 * M, CR * M)],
                             d_bufs[b], sems[b])

        def wait_chunk(b):
            pltpu.make_async_copy(nb_hbm.at[pl.ds(0, CR * PM)],
                                  nb_bufs[b], sems[b]).wait()
            pltpu.make_async_copy(d_hbm.at[pl.ds(0, CR * M)],
                                  d_bufs[b], sems[b]).wait()

        def process_chunk(c, b, acc):
            nb_v = nb_bufs[b]
            d_v = d_bufs[b]

            def row_body(r, acc):
                row_sum = jnp.zeros((_L,), jnp.float32)
                for v in range(NV):
                    idx = plsc.load_gather(
                        nb_v, [r * PM + (2 * v * _L) + 2 * iota])
                    qj = plsc.load_gather(q_v, [idx])
                    dv = plsc.load_gather(d_v, [r * M + v * _L + iota])
                    row_sum = row_sum + qj / dv
                qi = q_v[pl.ds(base_row + c * CR + r, _L)][0]
                return acc + qi * row_sum

            return lax.fori_loop(jnp.int32(0), jnp.int32(CR), row_body, acc,
                                 unroll=False)

        start_chunk(0, 0)
        pltpu.make_async_copy(q_hbm, q_v.at[pl.ds(0, N)], semt).wait()

        def body(k, acc):
            g = 2 * k
            start_chunk(g + 1, 1)
            wait_chunk(0)
            acc = process_chunk(g, 0, acc)
            start_chunk(g + 2, 0)
            wait_chunk(1)
            acc = process_chunk(g + 1, 1, acc)
            return acc

        acc = lax.fori_loop(jnp.int32(0), jnp.int32((NCH - 1) // 2),
                            body, jnp.zeros((_L,), jnp.float32))
        wait_chunk(0)
        acc = process_chunk(NCH - 1, 0, acc)

        acc_v[...] = acc
        pltpu.sync_copy(acc_v, out_hbm.at[wid])

    return sc_fn


@jax.jit
def kernel(charges, d_ij_lr, nbmat_lr):
    B, N, M = d_ij_lr.shape
    q = charges.reshape(N)
    d = d_ij_lr.reshape(N * M)
    nb = lax.bitcast_convert_type(nbmat_lr, jnp.int32).reshape(N * M * 2)
    partials = _make_sc_call(N, M)(q, d, nb)
    e = _FACTOR * jnp.sum(partials.astype(jnp.float64))
    return e.reshape(B)


# speedup vs baseline: 143.5642x; 19.4858x over previous
import functools

import jax
import jax.numpy as jnp
from jax import lax
from jax.experimental import pallas as pl
from jax.experimental.pallas import tpu as pltpu
from jax.experimental.pallas import tpu_sc as plsc

jax.config.update("jax_enable_x64", True)

_FACTOR = 13.605693122994 * 0.5291772105638411

_NC = 2
_NS = 16
_NW = _NC * _NS
_L = 16
_CE = 3200
_NCH = 64


def _make_sc_call(N, E_pad):
    EPW = E_pad // _NW
    assert EPW == _NCH * _CE

    mesh = plsc.VectorSubcoreMesh(core_axis_name="c", subcore_axis_name="s")

    @functools.partial(
        pl.kernel,
        out_type=jax.ShapeDtypeStruct((_NW, _L), jnp.float32),
        mesh=mesh,
        scratch_types=[
            pltpu.VMEM((N,), jnp.float32),
            pltpu.VMEM((_CE,), jnp.int32),
            pltpu.VMEM((_CE,), jnp.int32),
            pltpu.VMEM((_CE,), jnp.float32),
            pltpu.VMEM((_CE,), jnp.float32),
            pltpu.VMEM((_L,), jnp.float32),
            pltpu.SemaphoreType.DMA,
            pltpu.SemaphoreType.DMA,
            pltpu.SemaphoreType.DMA,
        ],
        compiler_params=pltpu.CompilerParams(needs_layout_passes=False),
    )
    def sc_fn(q_hbm, t_hbm, nb_hbm, out_hbm,
              q_v, nb_v0, nb_v1, t_v0, t_v1, acc_v, sem0, sem1, semt):
        wid = lax.axis_index("s") * _NC + lax.axis_index("c")
        base = wid * EPW
        nb_bufs = (nb_v0, nb_v1)
        t_bufs = (t_v0, t_v1)
        sems = (sem0, sem1)

        pltpu.async_copy(q_hbm, q_v, semt)

        def start_chunk(c, b):
            e0 = base + c * _CE
            pltpu.async_copy(nb_hbm.at[pl.ds(e0, _CE)], nb_bufs[b], sems[b])
            pltpu.async_copy(t_hbm.at[pl.ds(e0, _CE)], t_bufs[b], sems[b])

        def wait_chunk(b):
            pltpu.make_async_copy(nb_hbm.at[pl.ds(0, _CE)],
                                  nb_bufs[b], sems[b]).wait()
            pltpu.make_async_copy(t_hbm.at[pl.ds(0, _CE)],
                                  t_bufs[b], sems[b]).wait()

        def process_chunk(b, acc):
            nb_v = nb_bufs[b]
            t_v = t_bufs[b]

            def vec_body(v, acc):
                off = v * (4 * _L)
                for u in range(4):
                    idx = nb_v[pl.ds(off + u * _L, _L)]
                    tv = t_v[pl.ds(off + u * _L, _L)]
                    qj = plsc.load_gather(q_v, [idx])
                    acc = acc + qj * tv
                return acc

            return lax.fori_loop(jnp.int32(0), jnp.int32(_CE // (4 * _L)),
                                 vec_body, acc, unroll=False)

        start_chunk(0, 0)
        pltpu.make_async_copy(q_hbm, q_v, semt).wait()

        def body(k, acc):
            g = 2 * k
            start_chunk(g + 1, 1)
            wait_chunk(0)
            acc = process_chunk(0, acc)
            start_chunk(g + 2, 0)
            wait_chunk(1)
            acc = process_chunk(1, acc)
            return acc

        acc = lax.fori_loop(jnp.int32(0), jnp.int32(_NCH // 2 - 1),
                            body, jnp.zeros((_L,), jnp.float32))
        start_chunk(_NCH - 1, 1)
        wait_chunk(0)
        acc = process_chunk(0, acc)
        wait_chunk(1)
        acc = process_chunk(1, acc)

        acc_v[...] = acc
        pltpu.sync_copy(acc_v, out_hbm.at[wid])

    return sc_fn


@jax.jit
def kernel(charges, d_ij_lr, nbmat_lr):
    B, N, M = d_ij_lr.shape
    E = N * M
    E_pad = _NW * _NCH * _CE
    q = charges[0]
    t = (charges[:, :, None] / d_ij_lr).reshape(E)
    t = jnp.pad(t, (0, E_pad - E))
    nbl = lax.convert_element_type(nbmat_lr, jnp.int32).reshape(E)
    nbl = jnp.pad(nbl, (0, E_pad - E))
    partials = _make_sc_call(N, E_pad)(q, t, nbl)
    e = _FACTOR * jnp.sum(partials.astype(jnp.float64))
    return e.reshape(B)


# speedup vs baseline: 159.7075x; 1.1124x over previous
import functools

import jax
import jax.numpy as jnp
from jax import lax
from jax.experimental import pallas as pl
from jax.experimental.pallas import tpu as pltpu
from jax.experimental.pallas import tpu_sc as plsc

jax.config.update("jax_enable_x64", True)

_FACTOR = 13.605693122994 * 0.5291772105638411

_NC = 2
_NS = 16
_NW = _NC * _NS
_L = 16
_CE = 4000
_NCH = 50


def _make_sc_call(N, E_pad):
    EPW = E_pad // _NW
    assert EPW == _NCH * _CE

    mesh = plsc.VectorSubcoreMesh(core_axis_name="c", subcore_axis_name="s")

    @functools.partial(
        pl.kernel,
        out_type=jax.ShapeDtypeStruct((_NW, _L), jnp.float32),
        mesh=mesh,
        scratch_types=[
            pltpu.VMEM((N,), jnp.float32),
            pltpu.VMEM((_CE,), jnp.int32),
            pltpu.VMEM((_CE,), jnp.int32),
            pltpu.VMEM((_CE,), jnp.float32),
            pltpu.VMEM((_CE,), jnp.float32),
            pltpu.VMEM((_L,), jnp.float32),
            pltpu.SemaphoreType.DMA,
            pltpu.SemaphoreType.DMA,
            pltpu.SemaphoreType.DMA,
        ],
        compiler_params=pltpu.CompilerParams(needs_layout_passes=False),
    )
    def sc_fn(q_hbm, t_hbm, nb_hbm, out_hbm,
              q_v, nb_v0, nb_v1, t_v0, t_v1, acc_v, sem0, sem1, semt):
        wid = lax.axis_index("s") * _NC + lax.axis_index("c")
        base = wid * EPW
        nb_bufs = (nb_v0, nb_v1)
        t_bufs = (t_v0, t_v1)
        sems = (sem0, sem1)

        pltpu.async_copy(q_hbm, q_v, semt)

        def start_chunk(c, b):
            e0 = base + c * _CE
            pltpu.async_copy(nb_hbm.at[pl.ds(e0, _CE)], nb_bufs[b], sems[b])
            pltpu.async_copy(t_hbm.at[pl.ds(e0, _CE)], t_bufs[b], sems[b])

        def wait_chunk(b):
            pltpu.make_async_copy(nb_hbm.at[pl.ds(0, _CE)],
                                  nb_bufs[b], sems[b]).wait()
            pltpu.make_async_copy(t_hbm.at[pl.ds(0, _CE)],
                                  t_bufs[b], sems[b]).wait()

        def process_chunk(b, acc):
            nb_v = nb_bufs[b]
            t_v = t_bufs[b]

            def vec_body(v, acc):
                off = v * (5 * _L)
                for u in range(5):
                    idx = nb_v[pl.ds(off + u * _L, _L)]
                    tv = t_v[pl.ds(off + u * _L, _L)]
                    qj = plsc.load_gather(q_v, [idx])
                    acc = acc + qj * tv
                return acc

            return lax.fori_loop(jnp.int32(0), jnp.int32(_CE // (5 * _L)),
                                 vec_body, acc, unroll=False)

        start_chunk(0, 0)
        pltpu.make_async_copy(q_hbm, q_v, semt).wait()

        def body(k, acc):
            g = 2 * k
            start_chunk(g + 1, 1)
            wait_chunk(0)
            acc = process_chunk(0, acc)
            start_chunk(g + 2, 0)
            wait_chunk(1)
            acc = process_chunk(1, acc)
            return acc

        acc = lax.fori_loop(jnp.int32(0), jnp.int32(_NCH // 2 - 1),
                            body, jnp.zeros((_L,), jnp.float32))
        start_chunk(_NCH - 1, 1)
        wait_chunk(0)
        acc = process_chunk(0, acc)
        wait_chunk(1)
        acc = process_chunk(1, acc)

        acc_v[...] = acc
        pltpu.sync_copy(acc_v, out_hbm.at[wid])

    return sc_fn


@jax.jit
def kernel(charges, d_ij_lr, nbmat_lr):
    B, N, M = d_ij_lr.shape
    E = N * M
    assert E == _NW * _NCH * _CE
    q = charges[0]
    t = (charges[:, :, None] / d_ij_lr).reshape(E)
    nbl = lax.convert_element_type(nbmat_lr, jnp.int32).reshape(E)
    partials = _make_sc_call(N, E)(q, t, nbl)
    e = _FACTOR * jnp.sum(partials.astype(jnp.float64))
    return e.reshape(B)


# speedup vs baseline: 164.9868x; 1.0331x over previous
import functools

import jax
import jax.numpy as jnp
from jax import lax
from jax.experimental import pallas as pl
from jax.experimental.pallas import tpu as pltpu
from jax.experimental.pallas import tpu_sc as plsc

jax.config.update("jax_enable_x64", True)

_FACTOR = 13.605693122994 * 0.5291772105638411

_NC = 2
_NS = 16
_NW = _NC * _NS
_L = 16
_CE = 2000
_NCH = 100
_NB = 5


def _make_sc_call(N, E_pad):
    EPW = E_pad // _NW
    assert EPW == _NCH * _CE

    mesh = plsc.VectorSubcoreMesh(core_axis_name="c", subcore_axis_name="s")

    @functools.partial(
        pl.kernel,
        out_type=jax.ShapeDtypeStruct((_NW, _L), jnp.float32),
        mesh=mesh,
        scratch_types=(
            [pltpu.VMEM((N,), jnp.float32)]
            + [pltpu.VMEM((_CE,), jnp.int32) for _ in range(_NB)]
            + [pltpu.VMEM((_CE,), jnp.float32) for _ in range(_NB)]
            + [pltpu.VMEM((_L,), jnp.float32)]
            + [pltpu.SemaphoreType.DMA for _ in range(_NB)]
            + [pltpu.SemaphoreType.DMA]
        ),
        compiler_params=pltpu.CompilerParams(needs_layout_passes=False),
    )
    def sc_fn(q_hbm, t_hbm, nb_hbm, out_hbm, *refs):
        q_v = refs[0]
        nb_bufs = refs[1:1 + _NB]
        t_bufs = refs[1 + _NB:1 + 2 * _NB]
        acc_v = refs[1 + 2 * _NB]
        sems = refs[2 + 2 * _NB:2 + 3 * _NB]
        semt = refs[2 + 3 * _NB]
        wid = lax.axis_index("s") * _NC + lax.axis_index("c")
        base = wid * EPW

        off = pl.multiple_of(((wid * (N // _NW)) & ~7) + 8, 8)
        pltpu.async_copy(q_hbm.at[pl.ds(off, N - off)],
                         q_v.at[pl.ds(off, N - off)], semt)
        pltpu.async_copy(q_hbm.at[pl.ds(0, off)], q_v.at[pl.ds(0, off)], semt)

        def start_chunk(c, b):
            e0 = base + c * _CE
            pltpu.async_copy(nb_hbm.at[pl.ds(e0, _CE)], nb_bufs[b], sems[b])
            pltpu.async_copy(t_hbm.at[pl.ds(e0, _CE)], t_bufs[b], sems[b])

        def wait_chunk(b):
            pltpu.make_async_copy(nb_hbm.at[pl.ds(0, _CE)],
                                  nb_bufs[b], sems[b]).wait()
            pltpu.make_async_copy(t_hbm.at[pl.ds(0, _CE)],
                                  t_bufs[b], sems[b]).wait()

        def process_chunk(b, acc):
            nb_v = nb_bufs[b]
            t_v = t_bufs[b]

            def vec_body(v, acc):
                off = v * (5 * _L)
                for u in range(5):
                    idx = nb_v[pl.ds(off + u * _L, _L)]
                    tv = t_v[pl.ds(off + u * _L, _L)]
                    qj = plsc.load_gather(q_v, [idx])
                    acc = acc + qj * tv
                return acc

            return lax.fori_loop(jnp.int32(0), jnp.int32(_CE // (5 * _L)),
                                 vec_body, acc, unroll=False)

        for b in range(_NB):
            start_chunk(b, b)
        pltpu.make_async_copy(q_hbm.at[pl.ds(0, N - off)],
                              q_v.at[pl.ds(0, N - off)], semt).wait()
        pltpu.make_async_copy(q_hbm.at[pl.ds(0, off)],
                              q_v.at[pl.ds(0, off)], semt).wait()

        def body(k, acc):
            g = _NB * k
            for b in range(_NB):
                wait_chunk(b)
                acc = process_chunk(b, acc)
                start_chunk(g + _NB + b, b)
            return acc

        acc = lax.fori_loop(jnp.int32(0), jnp.int32(_NCH // _NB - 1),
                            body, jnp.zeros((_L,), jnp.float32))
        for b in range(_NB):
            wait_chunk(b)
            acc = process_chunk(b, acc)

        acc_v[...] = acc
        pltpu.sync_copy(acc_v, out_hbm.at[wid])

    return sc_fn


@jax.jit
def kernel(charges, d_ij_lr, nbmat_lr):
    B, N, M = d_ij_lr.shape
    E = N * M
    assert E == _NW * _NCH * _CE
    q = charges[0]
    t = (charges[:, :, None] / d_ij_lr).reshape(E)
    nbl = lax.convert_element_type(nbmat_lr, jnp.int32).reshape(E)
    partials = _make_sc_call(N, E)(q, t, nbl)
    e = _FACTOR * jnp.sum(partials.astype(jnp.float64))
    return e.reshape(B)


# speedup vs baseline: 165.0458x; 1.0004x over previous
import functools

import jax
import jax.numpy as jnp
from jax import lax
from jax.experimental import pallas as pl
from jax.experimental.pallas import tpu as pltpu
from jax.experimental.pallas import tpu_sc as plsc

jax.config.update("jax_enable_x64", True)

_FACTOR = 13.605693122994 * 0.5291772105638411

_NC = 2
_NS = 16
_NW = _NC * _NS
_L = 16
_CE = 2000
_NCH = 100
_NB = 5


def _make_sc_call(N, E):
    EPW = E // _NW
    assert EPW == _NCH * _CE

    mesh = plsc.VectorSubcoreMesh(core_axis_name="c", subcore_axis_name="s")

    @functools.partial(
        pl.kernel,
        out_type=jax.ShapeDtypeStruct((_NW, _L), jnp.float32),
        mesh=mesh,
        scratch_types=(
            [pltpu.VMEM((N,), jnp.float32)]
            + [pltpu.VMEM((_CE,), jnp.int32) for _ in range(_NB)]
            + [pltpu.VMEM((_CE,), jnp.float32) for _ in range(_NB)]
            + [pltpu.VMEM((_L,), jnp.float32)]
            + [pltpu.SemaphoreType.DMA for _ in range(_NB)]
            + [pltpu.SemaphoreType.DMA]
        ),
        compiler_params=pltpu.CompilerParams(needs_layout_passes=False),
    )
    def sc_fn(q_hbm, t_hbm, nb_hbm, out_hbm, *refs):
        q_v = refs[0]
        nb_bufs = refs[1:1 + _NB]
        t_bufs = refs[1 + _NB:1 + 2 * _NB]
        acc_v = refs[1 + 2 * _NB]
        sems = refs[2 + 2 * _NB:2 + 3 * _NB]
        semt = refs[2 + 3 * _NB]
        wid = lax.axis_index("s") * _NC + lax.axis_index("c")
        base = wid * EPW

        off = pl.multiple_of(((wid * (N // _NW)) & ~7) + 8, 8)
        pltpu.async_copy(q_hbm.at[pl.ds(off, N - off)],
                         q_v.at[pl.ds(off, N - off)], semt)
        pltpu.async_copy(q_hbm.at[pl.ds(0, off)], q_v.at[pl.ds(0, off)], semt)

        def start_chunk(c, b):
            e0 = base + c * _CE
            pltpu.async_copy(nb_hbm.at[pl.ds(e0, _CE)], nb_bufs[b], sems[b])
            pltpu.async_copy(t_hbm.at[pl.ds(e0, _CE)], t_bufs[b], sems[b])

        def wait_chunk(b):
            pltpu.make_async_copy(nb_hbm.at[pl.ds(0, _CE)],
                                  nb_bufs[b], sems[b]).wait()
            pltpu.make_async_copy(t_hbm.at[pl.ds(0, _CE)],
                                  t_bufs[b], sems[b]).wait()

        def process_chunk(b, acc):
            nb_v = nb_bufs[b]
            t_v = t_bufs[b]

            def vec_body(v, acc):
                off = v * (5 * _L)
                for u in range(5):
                    idx = nb_v[pl.ds(off + u * _L, _L)]
                    tv = t_v[pl.ds(off + u * _L, _L)]
                    qj = plsc.load_gather(q_v, [idx])
                    acc = acc + qj * tv
                return acc

            return lax.fori_loop(jnp.int32(0), jnp.int32(_CE // (5 * _L)),
                                 vec_body, acc, unroll=False)

        for b in range(_NB):
            start_chunk(b, b)
        pltpu.make_async_copy(q_hbm.at[pl.ds(0, N - off)],
                              q_v.at[pl.ds(0, N - off)], semt).wait()
        pltpu.make_async_copy(q_hbm.at[pl.ds(0, off)],
                              q_v.at[pl.ds(0, off)], semt).wait()

        def body(k, acc):
            g = _NB * k
            for b in range(_NB):
                wait_chunk(b)
                acc = process_chunk(b, acc)
                start_chunk(g + _NB + b, b)
            return acc

        acc = lax.fori_loop(jnp.int32(0), jnp.int32(_NCH // _NB - 1),
                            body, jnp.zeros((_L,), jnp.float32))
        for b in range(_NB):
            wait_chunk(b)
            acc = process_chunk(b, acc)

        acc_v[...] = acc
        pltpu.sync_copy(acc_v, out_hbm.at[wid])

    return sc_fn


@jax.jit
def kernel(charges, d_ij_lr, nbmat_lr):
    B, N, M = d_ij_lr.shape
    E = N * M
    assert E == _NW * _NCH * _CE
    q = charges[0]
    t = (charges[:, :, None] / d_ij_lr).reshape(E)
    nbl = lax.convert_element_type(nbmat_lr, jnp.int32).reshape(E)
    partials = _make_sc_call(N, E)(q, t, nbl)
    e = _FACTOR * jnp.sum(partials.astype(jnp.float64))
    return e.reshape(B)
